# baseline (speedup 1.0000x reference)
import functools

import numpy as np
import jax
import jax.numpy as jnp
from jax import lax
from jax.experimental import pallas as pl
from jax.experimental.pallas import tpu as pltpu
from jax.experimental.pallas import tpu_sc as plsc

_IN_DIM = 3
_OUT_DIM = 32
_BIN_NUM = 65536
_N_POINTS = 262144

_NC = 2
_NS = 16
_NW = _NC * _NS
_PPW = _N_POINTS // _NW
_CHUNK = 512
_NPAIR = _PPW // (2 * _CHUNK)
_VPA = _CHUNK // 16
_BURSTS = (_IN_DIM * _CHUNK) // 128
_BPA = _CHUNK // 128


def _body(x0_hbm, x1_hbm, x2_hbm, table_hbm, out_hbm,
          x0_v, x1_v, x2_v, idx_a, idx_b,
          ra0, ra1, ra2, rb0, rb1, rb2, o_a, o_b, gsem, osem):
    wid = lax.axis_index("s") * _NC + lax.axis_index("c")
    wbase = wid * _PPW

    xs = (x0_v, x1_v, x2_v)
    for a, xh in enumerate((x0_hbm, x1_hbm, x2_hbm)):
        pltpu.sync_copy(xh.at[pl.ds(wbase, _PPW)], xs[a])

    third = jnp.float32(1.0 / 3.0)

    def compute_ids(cbase, idx_v):
        for a in range(_IN_DIM):
            for v in range(_VPA):
                xv = xs[a][pl.ds(cbase + v * 16, 16)]
                idf = (xv + 1.0) * (0.5 * (_BIN_NUM - 1))
                ii = idf.astype(jnp.int32)
                ii = jnp.maximum(jnp.minimum(ii, _BIN_NUM - 1), 0)
                ii = ii + a * _BIN_NUM
                flat = a * _CHUNK + v * 16
                idx_v[flat // 128, pl.ds(flat % 128, 16)] = ii

    def each_gather(idx_v, rows):
        for a in range(_IN_DIM):
            for b in range(_BPA):
                yield (
                    table_hbm.at[idx_v.at[a * _BPA + b]],
                    rows[a].at[pl.ds(b * 128, 128)],
                )

    def fire_gathers(idx_v, rows):
        for src, dst in each_gather(idx_v, rows):
            pltpu.async_copy(src, dst, gsem)

    def wait_gathers(idx_v, rows):
        for src, dst in each_gather(idx_v, rows):
            pltpu.make_async_copy(src, dst, gsem).wait()

    def do_mean(r0, r1, r2, o_v):
        def mean_body(p, c2):
            for u in range(4):
                q = p * 4 + u
                a0, b0 = plsc.unpack(r0[q], format=plsc.PackFormat.INTERLEAVED)
                a1, b1 = plsc.unpack(r1[q], format=plsc.PackFormat.INTERLEAVED)
                a2, b2 = plsc.unpack(r2[q], format=plsc.PackFormat.INTERLEAVED)
                sa = (a0 + a1 + a2) * third
                sb = (b0 + b1 + b2) * third
                o_v[q, pl.ds(0, 16)] = sa
                o_v[q, pl.ds(16, 16)] = sb
            return c2

        lax.fori_loop(0, _CHUNK // 4, mean_body, 0, unroll=False)

    def fire_out(o_v, base):
        pltpu.async_copy(o_v, out_hbm.at[pl.ds(base, _CHUNK)], osem)

    def wait_out(o_v):
        pltpu.make_async_copy(o_v, out_hbm.at[pl.ds(0, _CHUNK)], osem).wait()

    compute_ids(0, idx_a)
    fire_gathers(idx_a, (ra0, ra1, ra2))

    def pair_body(k, carry):
        b0 = k * (2 * _CHUNK)
        b1 = b0 + _CHUNK

        wait_gathers(idx_a, (ra0, ra1, ra2))
        compute_ids(b1, idx_b)
        fire_gathers(idx_b, (rb0, rb1, rb2))

        @pl.when(k > 0)
        def _():
            wait_out(o_a)

        do_mean(ra0, ra1, ra2, o_a)
        fire_out(o_a, wbase + b0)

        wait_gathers(idx_b, (rb0, rb1, rb2))

        @pl.when(k < _NPAIR - 1)
        def _():
            compute_ids(b0 + 2 * _CHUNK, idx_a)
            fire_gathers(idx_a, (ra0, ra1, ra2))

        @pl.when(k > 0)
        def _():
            wait_out(o_b)

        do_mean(rb0, rb1, rb2, o_b)
        fire_out(o_b, wbase + b1)
        return carry

    lax.fori_loop(0, _NPAIR, pair_body, 0, unroll=False)

    wait_out(o_a)
    wait_out(o_b)


@jax.jit
def _run(x0, x1, x2, table_bf):
    mesh = plsc.VectorSubcoreMesh(core_axis_name="c", subcore_axis_name="s")
    f = pl.kernel(
        _body,
        out_type=jax.ShapeDtypeStruct((_N_POINTS, _OUT_DIM), jnp.float32),
        mesh=mesh,
        scratch_types=[
            pltpu.VMEM((_PPW,), jnp.float32),
            pltpu.VMEM((_PPW,), jnp.float32),
            pltpu.VMEM((_PPW,), jnp.float32),
            pltpu.VMEM((_BURSTS, 128), jnp.int32),
            pltpu.VMEM((_BURSTS, 128), jnp.int32),
            pltpu.VMEM((_CHUNK, _OUT_DIM), jnp.bfloat16),
            pltpu.VMEM((_CHUNK, _OUT_DIM), jnp.bfloat16),
            pltpu.VMEM((_CHUNK, _OUT_DIM), jnp.bfloat16),
            pltpu.VMEM((_CHUNK, _OUT_DIM), jnp.bfloat16),
            pltpu.VMEM((_CHUNK, _OUT_DIM), jnp.bfloat16),
            pltpu.VMEM((_CHUNK, _OUT_DIM), jnp.bfloat16),
            pltpu.VMEM((_CHUNK, _OUT_DIM), jnp.float32),
            pltpu.VMEM((_CHUNK, _OUT_DIM), jnp.float32),
            pltpu.SemaphoreType.DMA,
            pltpu.SemaphoreType.DMA,
        ],
        compiler_params=pltpu.CompilerParams(
            use_tc_tiling_on_sc=False, needs_layout_passes=False
        ),
    )
    return f(x0, x1, x2, table_bf)


_COL_PERM = np.empty(_OUT_DIM, dtype=np.int32)
_COL_PERM[0::2] = np.arange(_OUT_DIM // 2)
_COL_PERM[1::2] = np.arange(_OUT_DIM // 2) + _OUT_DIM // 2


def kernel(in_tensor, table):
    x_t = in_tensor.T
    table_bf = table.astype(jnp.bfloat16)[:, _COL_PERM]
    return _run(x_t[0], x_t[1], x_t[2], table_bf)
